# Initial kernel scaffold; baseline (speedup 1.0000x reference)
import jax
import jax.numpy as jnp
from jax.experimental import pallas as pl


def kernel(adj_indices, adj_values, embs):
    raise NotImplementedError("write your pallas kernel here")


# speedup vs baseline: 2.2524x; 2.2524x over previous
import jax
import jax.numpy as jnp
from jax import lax
from jax.experimental import pallas as pl
from jax.experimental.pallas import tpu as pltpu
from jax.experimental.pallas import tpu_sc as plsc

N = 10000
NNZ = 320000
D = 128
DH = 64
LEAKY = 0.2
NC = 2
NS = 16
CHUNK = 128
NCHUNKS = NNZ // CHUNK
CHUNKS_LO = NCHUNKS // NS
CHUNKS_REM = NCHUNKS % NS
GROUP = 40
NGROUPS = N // GROUP
GROUPS_LO = NGROUPS // NS
GROUPS_REM = NGROUPS % NS


def _body(row_hbm, col_hbm, val_hbm, emb_hbm, out_hbm,
          tmp_sp, acc_sp, idx_g, idx_s, val_v, rows_v, obuf, sem):
    c = lax.axis_index("c")
    s = lax.axis_index("s")

    def _zero_row(r, carry):
        for k in range(DH // 16):
            obuf[r, pl.ds(k * 16, 16)] = jnp.zeros((16,), jnp.float32)
        return carry
    lax.fori_loop(0, GROUP, _zero_row, 0)
    n_groups = GROUPS_LO + jnp.where(s < GROUPS_REM, 1, 0)

    def _zero_group(i, carry):
        g0 = (i * NS + s) * GROUP
        pltpu.sync_copy(obuf, tmp_sp.at[pl.ds(g0, GROUP)])
        pltpu.sync_copy(obuf, acc_sp.at[pl.ds(g0, GROUP)])
        return carry
    lax.fori_loop(0, n_groups, _zero_group, 0)
    plsc.subcore_barrier()

    base_chunk = s * CHUNKS_LO + jnp.minimum(s, CHUNKS_REM)
    n_chunks = CHUNKS_LO + jnp.where(s < CHUNKS_REM, 1, 0)

    def _scale_rows():
        def _group(j, carry):
            v16 = val_v[pl.ds(j * 16, 16)]
            for lane in range(16):
                e = j * 16 + lane
                sv = v16[lane]
                for k in range(DH // 16):
                    sl = rows_v[e, pl.ds(k * 16, 16)]
                    rows_v[e, pl.ds(k * 16, 16)] = sl * sv
            return carry
        lax.fori_loop(0, CHUNK // 16, _group, 0)

    def _p1(i, carry):
        off = (base_chunk + i) * CHUNK
        pltpu.sync_copy(row_hbm.at[pl.ds(off, CHUNK)], idx_g)
        pltpu.sync_copy(col_hbm.at[pl.ds(off, CHUNK)], idx_s)
        pltpu.sync_copy(val_hbm.at[pl.ds(off, CHUNK)], val_v)
        coff = (c * N).astype(jnp.int32)
        for k in range(CHUNK // 16):
            idx_g[pl.ds(k * 16, 16)] = idx_g[pl.ds(k * 16, 16)] + coff
        pltpu.async_copy(emb_hbm.at[idx_g], rows_v, sem).wait()
        _scale_rows()
        pltpu.sync_copy(rows_v, tmp_sp.at[idx_s], add=True)
        return carry
    lax.fori_loop(0, n_chunks, _p1, 0)
    plsc.subcore_barrier()

    def _p2(i, carry):
        off = (base_chunk + i) * CHUNK
        pltpu.sync_copy(col_hbm.at[pl.ds(off, CHUNK)], idx_g)
        pltpu.sync_copy(row_hbm.at[pl.ds(off, CHUNK)], idx_s)
        pltpu.sync_copy(val_hbm.at[pl.ds(off, CHUNK)], val_v)
        pltpu.async_copy(tmp_sp.at[idx_g], rows_v, sem).wait()
        _scale_rows()
        pltpu.sync_copy(rows_v, acc_sp.at[idx_s], add=True)
        return carry
    lax.fori_loop(0, n_chunks, _p2, 0)
    plsc.subcore_barrier()

    def _act_group(i, carry):
        g0 = (i * NS + s) * GROUP
        pltpu.sync_copy(acc_sp.at[pl.ds(g0, GROUP)], obuf)
        def _act_row(r, inner):
            for k in range(DH // 16):
                x = obuf[r, pl.ds(k * 16, 16)]
                obuf[r, pl.ds(k * 16, 16)] = jnp.maximum(x, x * LEAKY)
            return inner
        lax.fori_loop(0, GROUP, _act_row, 0)
        pltpu.sync_copy(obuf, out_hbm.at[c, pl.ds(g0, GROUP)])
        return carry
    lax.fori_loop(0, n_groups, _act_group, 0)


_sc_call = pl.kernel(
    _body,
    out_type=jax.ShapeDtypeStruct((NC, N, DH), jnp.float32),
    mesh=plsc.VectorSubcoreMesh(core_axis_name="c", subcore_axis_name="s"),
    compiler_params=pltpu.CompilerParams(use_tc_tiling_on_sc=False),
    scratch_types=[
        pltpu.VMEM_SHARED((N, DH), jnp.float32),
        pltpu.VMEM_SHARED((N, DH), jnp.float32),
        pltpu.VMEM((CHUNK,), jnp.int32),
        pltpu.VMEM((CHUNK,), jnp.int32),
        pltpu.VMEM((CHUNK,), jnp.float32),
        pltpu.VMEM((CHUNK, DH), jnp.float32),
        pltpu.VMEM((GROUP, DH), jnp.float32),
        pltpu.SemaphoreType.DMA,
    ],
)


@jax.jit
def kernel(adj_indices, adj_values, embs):
    row = adj_indices[0].astype(jnp.int32)
    col = adj_indices[1].astype(jnp.int32)
    emb2 = jnp.concatenate([embs[:, :DH], embs[:, DH:]], axis=0)
    out2 = _sc_call(row, col, adj_values, emb2)
    return jnp.concatenate([out2[0], out2[1]], axis=1)


# speedup vs baseline: 3.2025x; 1.4218x over previous
import jax
import jax.numpy as jnp
from jax import lax
from jax.experimental import pallas as pl
from jax.experimental.pallas import tpu as pltpu
from jax.experimental.pallas import tpu_sc as plsc

N = 10000
NNZ = 320000
D = 128
DH = 64
LEAKY = 0.2
NC = 2
NS = 16
CHUNK = 128
NCHUNKS = NNZ // CHUNK
CHUNKS_LO = NCHUNKS // NS
CHUNKS_REM = NCHUNKS % NS
GROUP = 40
NGROUPS = N // GROUP
GROUPS_LO = NGROUPS // NS
GROUPS_REM = NGROUPS % NS


def _body(p1_hbm, p2_hbm, emb_hbm, out_hbm,
          tmp_sp, acc_sp, pbuf, rows_v, obuf, gsem):
    c = lax.axis_index("c")
    s = lax.axis_index("s")

    def _zero_row(r, carry):
        for k in range(DH // 16):
            obuf[r, pl.ds(k * 16, 16)] = jnp.zeros((16,), jnp.float32)
        return carry
    lax.fori_loop(0, GROUP, _zero_row, 0)
    n_groups = GROUPS_LO + jnp.where(s < GROUPS_REM, 1, 0)

    def _zero_group(i, carry):
        g0 = (i * NS + s) * GROUP
        pltpu.sync_copy(obuf, tmp_sp.at[pl.ds(g0, GROUP)])
        pltpu.sync_copy(obuf, acc_sp.at[pl.ds(g0, GROUP)])
        return carry
    lax.fori_loop(0, n_groups, _zero_group, 0)
    plsc.subcore_barrier()

    base_chunk = s * CHUNKS_LO + jnp.minimum(s, CHUNKS_REM)
    n_chunks = CHUNKS_LO + jnp.where(s < CHUNKS_REM, 1, 0)

    def _scale_rows(ib):
        def _group(j, carry):
            bits = pbuf[ib, 2, pl.ds(j * 16, 16)]
            v16 = plsc.bitcast(bits, jnp.float32)
            for lane in range(16):
                e = j * 16 + lane
                sv = v16[lane]
                for k in range(DH // 16):
                    sl = rows_v[ib, e, pl.ds(k * 16, 16)]
                    rows_v[ib, e, pl.ds(k * 16, 16)] = sl * sv
            return carry
        lax.fori_loop(0, CHUNK // 16, _group, 0)

    def _run_phase(get_meta, gather_src, scatter_dst):
        def _start(i):
            ib = lax.rem(i, 2)
            pltpu.sync_copy(get_meta(i), pbuf.at[ib])
            pltpu.make_async_copy(
                gather_src.at[pbuf.at[ib, 0]], rows_v.at[ib], gsem.at[ib]
            ).start()

        def _finish(i):
            ib = lax.rem(i, 2)
            pltpu.make_async_copy(
                emb_hbm.at[pbuf.at[ib, 0]], rows_v.at[ib], gsem.at[ib]
            ).wait()
            _scale_rows(ib)
            pltpu.sync_copy(rows_v.at[ib], scatter_dst.at[pbuf.at[ib, 1]],
                            add=True)

        _start(0)
        def _step(i, carry):
            @pl.when(i + 1 < n_chunks)
            def _():
                _start(i + 1)
            _finish(i)
            return carry
        lax.fori_loop(0, n_chunks, _step, 0)

    _run_phase(lambda i: p1_hbm.at[c, base_chunk + i], emb_hbm, tmp_sp)
    plsc.subcore_barrier()

    _run_phase(lambda i: p2_hbm.at[base_chunk + i], tmp_sp, acc_sp)
    plsc.subcore_barrier()

    def _act_group(i, carry):
        g0 = (i * NS + s) * GROUP
        pltpu.sync_copy(acc_sp.at[pl.ds(g0, GROUP)], obuf)
        def _act_row(r, inner):
            for k in range(DH // 16):
                x = obuf[r, pl.ds(k * 16, 16)]
                obuf[r, pl.ds(k * 16, 16)] = jnp.maximum(x, x * LEAKY)
            return inner
        lax.fori_loop(0, GROUP, _act_row, 0)
        pltpu.sync_copy(obuf, out_hbm.at[c, pl.ds(g0, GROUP)])
        return carry
    lax.fori_loop(0, n_groups, _act_group, 0)


_sc_call = pl.kernel(
    _body,
    out_type=jax.ShapeDtypeStruct((NC, N, DH), jnp.float32),
    mesh=plsc.VectorSubcoreMesh(core_axis_name="c", subcore_axis_name="s"),
    compiler_params=pltpu.CompilerParams(use_tc_tiling_on_sc=False,
                                         needs_layout_passes=False),
    scratch_types=[
        pltpu.VMEM_SHARED((N, DH), jnp.float32),
        pltpu.VMEM_SHARED((N, DH), jnp.float32),
        pltpu.VMEM((2, 3, CHUNK), jnp.int32),
        pltpu.VMEM((2, CHUNK, DH), jnp.float32),
        pltpu.VMEM((GROUP, DH), jnp.float32),
        pltpu.SemaphoreType.DMA((2,)),
    ],
)


@jax.jit
def kernel(adj_indices, adj_values, embs):
    row = adj_indices[0].astype(jnp.int32)
    col = adj_indices[1].astype(jnp.int32)
    emb2 = jnp.concatenate([embs[:, :DH], embs[:, DH:]], axis=0)
    colr = col.reshape(NCHUNKS, CHUNK)
    rowr = row.reshape(NCHUNKS, CHUNK)
    bits = lax.bitcast_convert_type(adj_values, jnp.int32).reshape(
        NCHUNKS, CHUNK)
    p1 = jnp.stack([
        jnp.stack([rowr + cc * N, colr, bits], axis=1) for cc in range(NC)
    ])
    p2 = jnp.stack([colr, rowr, bits], axis=1)
    out2 = _sc_call(p1, p2, emb2)
    return jnp.concatenate([out2[0], out2[1]], axis=1)
